# Initial kernel scaffold; baseline (speedup 1.0000x reference)
import jax
import jax.numpy as jnp
from jax.experimental import pallas as pl


def kernel(p, n, x, o, params):
    raise NotImplementedError("write your pallas kernel here")


# speedup vs baseline: 3.3348x; 3.3348x over previous
import functools

import jax
import jax.numpy as jnp
from jax import lax
from jax.experimental import pallas as pl
from jax.experimental.pallas import tpu as pltpu
from jax.experimental.pallas import tpu_sc as plsc

N = 10000
NSAMP = 16
C = 128
SHARE = 8
CG = C // SHARE
TW = C + 16
EPS = 1e-5

KPAD = 10240
QB = 256
NQB = KPAD // QB

NB = 200
NBLK = N // NB
ER = NB * NSAMP

BIGF = 3.0e38
BIGI = 2 ** 30


def _knn_body(q_ref, kt_ref, qn_ref, kn_ref, idx_ref, dist_ref):
    cross = jnp.dot(q_ref[...], kt_ref[...],
                    preferred_element_type=jnp.float32)
    d2 = (qn_ref[...][:, 0:1] + kn_ref[...][0:1, :]) - 2.0 * cross
    iot = lax.broadcasted_iota(jnp.int32, (QB, KPAD), 1)
    idxs, dists = [], []
    for _ in range(NSAMP):
        m = jnp.min(d2, axis=1, keepdims=True)
        sel = jnp.min(jnp.where(d2 == m, iot, BIGI), axis=1,
                      keepdims=True)
        idxs.append(sel)
        dists.append(jnp.sqrt(jnp.maximum(m, 0.0)))
        d2 = jnp.where(iot == sel, BIGF, d2)
    idx_ref[...] = jnp.concatenate(idxs, axis=1)
    dist_ref[...] = jnp.concatenate(dists, axis=1)


def _knn(p):
    qpad = jnp.zeros((KPAD, 8), dtype=jnp.float32)
    qpad = qpad.at[:N, :3].set(p)
    qpad = qpad.at[N:, :3].set(1e15)
    kt = qpad.T[:8, :]
    nrm = jnp.sum(qpad[:, :3] * qpad[:, :3], axis=1)
    qn = jnp.broadcast_to(nrm[:, None], (KPAD, 8))
    kn = jnp.broadcast_to(nrm[None, :], (8, KPAD))
    idx, dist = pl.pallas_call(
        _knn_body,
        grid=(NQB,),
        in_specs=[
            pl.BlockSpec((QB, 8), lambda i: (i, 0)),
            pl.BlockSpec((8, KPAD), lambda i: (0, 0)),
            pl.BlockSpec((QB, 8), lambda i: (i, 0)),
            pl.BlockSpec((8, KPAD), lambda i: (0, 0)),
        ],
        out_specs=[
            pl.BlockSpec((QB, NSAMP), lambda i: (i, 0)),
            pl.BlockSpec((QB, NSAMP), lambda i: (i, 0)),
        ],
        out_shape=[
            jax.ShapeDtypeStruct((KPAD, NSAMP), jnp.int32),
            jax.ShapeDtypeStruct((KPAD, NSAMP), jnp.float32),
        ],
    )(qpad, kt, qn, kn)
    return idx[:N], dist[:N]


NW = 32
GCH = 128
GIT = 40
BPW = GCH * GIT
BPAD = NW * BPW


def _gather(table, idx_flat):
    idx_flat = jnp.pad(idx_flat, (0, BPAD - N * NSAMP))
    mesh = plsc.VectorSubcoreMesh(core_axis_name="c", subcore_axis_name="s")

    @functools.partial(
        pl.kernel,
        mesh=mesh,
        compiler_params=pltpu.CompilerParams(use_tc_tiling_on_sc=False),
        out_type=jax.ShapeDtypeStruct((BPAD, TW), jnp.float32),
        scratch_types=[
            pltpu.VMEM((GCH,), jnp.int32),
            pltpu.VMEM((GCH, TW), jnp.float32),
            pltpu.SemaphoreType.DMA,
        ],
    )
    def gather_k(table_hbm, idx_hbm, out_hbm, idx_v, rows_v, sem):
        wid = lax.axis_index("s") * 2 + lax.axis_index("c")
        base = wid * BPW

        def body(i, carry):
            off = base + i * GCH
            pltpu.sync_copy(idx_hbm.at[pl.ds(off, GCH)], idx_v)
            pltpu.async_copy(table_hbm.at[idx_v], rows_v, sem).wait()
            pltpu.sync_copy(rows_v, out_hbm.at[pl.ds(off, GCH)])
            return carry

        lax.fori_loop(0, GIT, body, 0)

    return gather_k(table, idx_flat)[:N * NSAMP]


def _p0_body(xgp_ref, p_ref, pa_ref, acc_ref):
    i = pl.program_id(0)
    pr3 = xgp_ref[...][:, :, :3] - p_ref[...][:, None, :3]
    pa = pa_ref[...]
    t = pa[3:4, :3].reshape(1, 1, 3)
    for j in range(3):
        t = t + pr3[:, :, j:j + 1] * pa[j:j + 1, :3].reshape(1, 1, 3)
    s1 = jnp.sum(t, axis=(0, 1)).reshape(1, 3)
    s2 = jnp.sum(t * t, axis=(0, 1)).reshape(1, 3)
    upd = jnp.concatenate(
        [jnp.pad(s1, ((0, 0), (0, 125))),
         jnp.pad(s2, ((0, 0), (0, 125))),
         jnp.zeros((6, 128), jnp.float32)], axis=0)

    @pl.when(i == 0)
    def _():
        acc_ref[...] = upd

    @pl.when(i > 0)
    def _():
        acc_ref[...] += upd


def _p0(xgp, p, pa):
    return pl.pallas_call(
        _p0_body,
        grid=(NBLK,),
        in_specs=[
            pl.BlockSpec((NB, NSAMP, 16), lambda i: (i, 0, 0)),
            pl.BlockSpec((NB, 8), lambda i: (i, 0)),
            pl.BlockSpec((8, 8), lambda i: (0, 0)),
        ],
        out_specs=pl.BlockSpec((8, 128), lambda i: (0, 0)),
        out_shape=jax.ShapeDtypeStruct((8, 128), jnp.float32),
    )(xgp, jnp.pad(p, ((0, 0), (0, 5))), pa)


def _p1_body(xg_ref, p_ref, x_ref, dist_ref, wqt_ref, wkt_ref, wvt_ref,
             bq_ref, bk_ref, bv_ref, pa_ref, wp2t_ref, bp2_ref,
             w_ref, veff_ref, acc_ref):
    i = pl.program_id(0)
    xg = xg_ref[...]
    xx = x_ref[...]
    e2 = (xg[:, :, :C] - xx[:, None, :]).reshape(ER, C)
    kk = jnp.dot(e2, wkt_ref[...],
                 preferred_element_type=jnp.float32) + bk_ref[...]
    vv = jnp.dot(e2, wvt_ref[...],
                 preferred_element_type=jnp.float32) + bv_ref[...]
    q = jnp.dot(xx, wqt_ref[...],
                preferred_element_type=jnp.float32) + bq_ref[...]

    pa = pa_ref[...]
    pr3 = xg[:, :, C:C + 3] - p_ref[...][:, None, :3]
    t = pa[3:4, :3].reshape(1, 1, 3)
    for j in range(3):
        t = t + pr3[:, :, j:j + 1] * pa[j:j + 1, :3].reshape(1, 1, 3)
    t = jnp.maximum(t, 0.0)
    pr = bp2_ref[...][None]
    for j in range(3):
        pr = pr + t[:, :, j:j + 1] * wp2t_ref[j:j + 1, :][None]
    w3 = q[:, None, :] - kk.reshape(NB, NSAMP, C) + pr
    dw = jnp.exp(-dist_ref[...])[:, :, None]
    veff = vv.reshape(NB, NSAMP, C) * dw + pr
    w_ref[...] = w3
    veff_ref[...] = veff
    sw = jnp.sum(w3, axis=(0, 1)).reshape(1, C)
    swsq = jnp.sum(w3 * w3, axis=(0, 1)).reshape(1, C)
    upd = jnp.concatenate([sw, swsq, jnp.zeros((6, C), jnp.float32)], axis=0)

    @pl.when(i == 0)
    def _():
        acc_ref[...] = upd

    @pl.when(i > 0)
    def _():
        acc_ref[...] += upd


def _p1(xg3, p, x, dist, params, pa):
    full = lambda r, c: pl.BlockSpec((r, c), lambda i: (0, 0))
    return pl.pallas_call(
        _p1_body,
        grid=(NBLK,),
        in_specs=[
            pl.BlockSpec((NB, NSAMP, TW), lambda i: (i, 0, 0)),
            pl.BlockSpec((NB, 8), lambda i: (i, 0)),
            pl.BlockSpec((NB, C), lambda i: (i, 0)),
            pl.BlockSpec((NB, NSAMP), lambda i: (i, 0)),
            full(C, C), full(C, C), full(C, C),
            full(1, C), full(1, C), full(1, C),
            full(8, 8), full(8, C), full(1, C),
        ],
        out_specs=[
            pl.BlockSpec((NB, NSAMP, C), lambda i: (i, 0, 0)),
            pl.BlockSpec((NB, NSAMP, C), lambda i: (i, 0, 0)),
            pl.BlockSpec((8, C), lambda i: (0, 0)),
        ],
        out_shape=[
            jax.ShapeDtypeStruct((N, NSAMP, C), jnp.float32),
            jax.ShapeDtypeStruct((N, NSAMP, C), jnp.float32),
            jax.ShapeDtypeStruct((8, C), jnp.float32),
        ],
    )(xg3, jnp.pad(p, ((0, 0), (0, 5))), x, dist,
      params['Wq'].T, params['Wk'].T, params['Wv'].T,
      params['bq'][None], params['bk'][None], params['bv'][None],
      pa, jnp.pad(params['Wp2'].T, ((0, 5), (0, 0))), params['bp2'][None])


def _p2_body(w_ref, aff_ref, wl1t_ref, bl1_ref, u_ref, acc_ref):
    i = pl.program_id(0)
    a = aff_ref[...]
    w = w_ref[...].reshape(ER, C)
    wb = jnp.maximum(w * a[0:1, :] + a[1:2, :], 0.0)
    u = jnp.dot(wb, wl1t_ref[...],
                preferred_element_type=jnp.float32) + bl1_ref[...]
    u_ref[...] = u.reshape(NB, NSAMP, CG)
    su = jnp.sum(u, axis=0).reshape(1, CG)
    susq = jnp.sum(u * u, axis=0).reshape(1, CG)
    upd = jnp.concatenate([su, susq, jnp.zeros((6, CG), jnp.float32)], axis=0)

    @pl.when(i == 0)
    def _():
        acc_ref[...] = upd

    @pl.when(i > 0)
    def _():
        acc_ref[...] += upd


def _p2(w, aff1, params):
    full = lambda r, c: pl.BlockSpec((r, c), lambda i: (0, 0))
    return pl.pallas_call(
        _p2_body,
        grid=(NBLK,),
        in_specs=[
            pl.BlockSpec((NB, NSAMP, C), lambda i: (i, 0, 0)),
            full(8, C), full(C, CG), full(1, CG),
        ],
        out_specs=[
            pl.BlockSpec((NB, NSAMP, CG), lambda i: (i, 0, 0)),
            pl.BlockSpec((8, CG), lambda i: (0, 0)),
        ],
        out_shape=[
            jax.ShapeDtypeStruct((N, NSAMP, CG), jnp.float32),
            jax.ShapeDtypeStruct((8, CG), jnp.float32),
        ],
    )(w, aff1, params['Wl1'].T, params['bl1'][None])


def _p3_body(u_ref, veff_ref, aff_ref, wl2t_ref, bl2_ref, out_ref):
    a = aff_ref[...]
    u = u_ref[...].reshape(ER, CG)
    ub = jnp.maximum(u * a[0:1, :] + a[1:2, :], 0.0)
    s = jnp.dot(ub, wl2t_ref[...],
                preferred_element_type=jnp.float32) + bl2_ref[...]
    s3 = s.reshape(NB, NSAMP, CG)
    mx = jnp.max(s3, axis=1, keepdims=True)
    es = jnp.exp(s3 - mx)
    sm = es / jnp.sum(es, axis=1, keepdims=True)
    v3 = veff_ref[...]
    outs = []
    for g in range(SHARE):
        outs.append(jnp.sum(v3[:, :, g * CG:(g + 1) * CG] * sm, axis=1))
    out_ref[...] = jnp.concatenate(outs, axis=1)


def _p3(u, veff, aff2, params):
    full = lambda r, c: pl.BlockSpec((r, c), lambda i: (0, 0))
    return pl.pallas_call(
        _p3_body,
        grid=(NBLK,),
        in_specs=[
            pl.BlockSpec((NB, NSAMP, CG), lambda i: (i, 0, 0)),
            pl.BlockSpec((NB, NSAMP, C), lambda i: (i, 0, 0)),
            full(8, CG), full(CG, CG), full(1, CG),
        ],
        out_specs=pl.BlockSpec((NB, C), lambda i: (i, 0)),
        out_shape=jax.ShapeDtypeStruct((N, C), jnp.float32),
    )(u, veff, aff2, params['Wl2'].T, params['bl2'][None])


def _bn_affine(ssum, ssq, cnt, gamma, beta):
    mean = ssum / cnt
    var = ssq / cnt - mean * mean
    scale = gamma / jnp.sqrt(var + EPS)
    return scale, beta - mean * scale


def kernel(p, n, x, o, params):
    idx, dist = _knn(p)
    table = jnp.concatenate([x, jnp.pad(p, ((0, 0), (0, 13)))], axis=1)
    xg3 = _gather(table, idx.reshape(-1)).reshape(N, NSAMP, TW)

    cnt = jnp.float32(N * NSAMP)
    pa0 = jnp.zeros((8, 8), jnp.float32)
    pa0 = pa0.at[:3, :3].set(params['Wp1'].T)
    pa0 = pa0.at[3, :3].set(params['bp1'])
    acc0 = _p0(xg3[:, :, C:], p, pa0)
    sc_p, sh_p = _bn_affine(acc0[0, :3], acc0[1, :3], cnt,
                            params['gp'], params['bp'])
    pa = jnp.zeros((8, 8), jnp.float32)
    pa = pa.at[:3, :3].set(params['Wp1'].T * sc_p[None, :])
    pa = pa.at[3, :3].set(params['bp1'] * sc_p + sh_p)

    w, veff, acc1 = _p1(xg3, p, x, dist, params, pa)
    sc1, sh1 = _bn_affine(acc1[0], acc1[1], cnt, params['g1'], params['b1'])
    aff1 = jnp.concatenate([sc1[None], sh1[None],
                            jnp.zeros((6, C), jnp.float32)], axis=0)

    u, acc2 = _p2(w, aff1, params)
    sc2, sh2 = _bn_affine(acc2[0], acc2[1], cnt, params['g2'], params['b2'])
    aff2 = jnp.concatenate([sc2[None], sh2[None],
                            jnp.zeros((6, CG), jnp.float32)], axis=0)

    return _p3(u, veff, aff2, params)


# speedup vs baseline: 3.4171x; 1.0247x over previous
import functools

import jax
import jax.numpy as jnp
from jax import lax
from jax.experimental import pallas as pl
from jax.experimental.pallas import tpu as pltpu
from jax.experimental.pallas import tpu_sc as plsc

N = 10000
NSAMP = 16
C = 128
SHARE = 8
CG = C // SHARE
TW = C + 16
EPS = 1e-5

KPAD = 10240
QB = 256
NQB = KPAD // QB

NB = 200
NBLK = N // NB
ER = NB * NSAMP

BIGF = 3.0e38
BIGI = 2 ** 30


def _knn_body(q_ref, kt_ref, qn_ref, kn_ref, idx_ref, dist_ref):
    cross = jnp.dot(q_ref[...], kt_ref[...],
                    preferred_element_type=jnp.float32)
    d2 = (qn_ref[...][:, 0:1] + kn_ref[...][0:1, :]) - 2.0 * cross
    idxs, dists = [], []
    for _ in range(NSAMP):
        m = jnp.min(d2, axis=1, keepdims=True)
        sel = jnp.argmin(d2, axis=1).astype(jnp.int32)[:, None]
        idxs.append(sel)
        dists.append(jnp.sqrt(jnp.maximum(m, 0.0)))
        iot = lax.broadcasted_iota(jnp.int32, (QB, KPAD), 1)
        d2 = jnp.where(iot == sel, BIGF, d2)
    idx_ref[...] = jnp.concatenate(idxs, axis=1)
    dist_ref[...] = jnp.concatenate(dists, axis=1)


def _knn(p):
    qpad = jnp.zeros((KPAD, 8), dtype=jnp.float32)
    qpad = qpad.at[:N, :3].set(p)
    qpad = qpad.at[N:, :3].set(1e15)
    kt = qpad.T[:8, :]
    nrm = jnp.sum(qpad[:, :3] * qpad[:, :3], axis=1)
    qn = jnp.broadcast_to(nrm[:, None], (KPAD, 8))
    kn = jnp.broadcast_to(nrm[None, :], (8, KPAD))
    idx, dist = pl.pallas_call(
        _knn_body,
        grid=(NQB,),
        in_specs=[
            pl.BlockSpec((QB, 8), lambda i: (i, 0)),
            pl.BlockSpec((8, KPAD), lambda i: (0, 0)),
            pl.BlockSpec((QB, 8), lambda i: (i, 0)),
            pl.BlockSpec((8, KPAD), lambda i: (0, 0)),
        ],
        out_specs=[
            pl.BlockSpec((QB, NSAMP), lambda i: (i, 0)),
            pl.BlockSpec((QB, NSAMP), lambda i: (i, 0)),
        ],
        out_shape=[
            jax.ShapeDtypeStruct((KPAD, NSAMP), jnp.int32),
            jax.ShapeDtypeStruct((KPAD, NSAMP), jnp.float32),
        ],
    )(qpad, kt, qn, kn)
    return idx[:N], dist[:N]


NW = 32
GCH = 128
GIT = 40
BPW = GCH * GIT
BPAD = NW * BPW


NRING = 4


def _gather(table, idx_flat):
    idx_flat = jnp.pad(idx_flat, (0, BPAD - N * NSAMP))
    mesh = plsc.VectorSubcoreMesh(core_axis_name="c", subcore_axis_name="s")

    @functools.partial(
        pl.kernel,
        mesh=mesh,
        compiler_params=pltpu.CompilerParams(use_tc_tiling_on_sc=False),
        out_type=jax.ShapeDtypeStruct((BPAD, TW), jnp.float32),
        scratch_types=[
            pltpu.VMEM((BPW,), jnp.int32),
        ] + [pltpu.VMEM((GCH, TW), jnp.float32) for _ in range(NRING)]
          + [pltpu.SemaphoreType.DMA for _ in range(2 * NRING)],
    )
    def gather_k(table_hbm, idx_hbm, out_hbm, idx_all, *bufs_sems):
        rows = bufs_sems[:NRING]
        gsem = bufs_sems[NRING:2 * NRING]
        wsem = bufs_sems[2 * NRING:]
        wid = lax.axis_index("s") * 2 + lax.axis_index("c")
        base = wid * BPW
        pltpu.sync_copy(idx_hbm.at[pl.ds(base, BPW)], idx_all)

        def body(t, carry):
            @pl.when(t > 0)
            def _():
                for b in range(NRING):
                    pltpu.make_async_copy(
                        rows[b], out_hbm.at[pl.ds(base, GCH)], wsem[b]).wait()

            for b in range(NRING):
                i = t * NRING + b
                pltpu.async_copy(
                    table_hbm.at[idx_all.at[pl.ds(i * GCH, GCH)]],
                    rows[b], gsem[b])
            for b in range(NRING):
                i = t * NRING + b
                pltpu.make_async_copy(
                    table_hbm.at[idx_all.at[pl.ds(i * GCH, GCH)]],
                    rows[b], gsem[b]).wait()
                pltpu.async_copy(
                    rows[b], out_hbm.at[pl.ds(base + i * GCH, GCH)], wsem[b])
            return carry

        lax.fori_loop(0, GIT // NRING, body, 0)
        for b in range(NRING):
            pltpu.make_async_copy(
                rows[b], out_hbm.at[pl.ds(base, GCH)], wsem[b]).wait()

    return gather_k(table, idx_flat)[:N * NSAMP]


def _p0_body(xgp_ref, p_ref, pa_ref, acc_ref):
    i = pl.program_id(0)
    pr3 = xgp_ref[...][:, :, :3] - p_ref[...][:, None, :3]
    pa = pa_ref[...]
    t = pa[3:4, :3].reshape(1, 1, 3)
    for j in range(3):
        t = t + pr3[:, :, j:j + 1] * pa[j:j + 1, :3].reshape(1, 1, 3)
    s1 = jnp.sum(t, axis=(0, 1)).reshape(1, 3)
    s2 = jnp.sum(t * t, axis=(0, 1)).reshape(1, 3)
    upd = jnp.concatenate(
        [jnp.pad(s1, ((0, 0), (0, 125))),
         jnp.pad(s2, ((0, 0), (0, 125))),
         jnp.zeros((6, 128), jnp.float32)], axis=0)

    @pl.when(i == 0)
    def _():
        acc_ref[...] = upd

    @pl.when(i > 0)
    def _():
        acc_ref[...] += upd


def _p0(xgp, p, pa):
    return pl.pallas_call(
        _p0_body,
        grid=(NBLK,),
        in_specs=[
            pl.BlockSpec((NB, NSAMP, 16), lambda i: (i, 0, 0)),
            pl.BlockSpec((NB, 8), lambda i: (i, 0)),
            pl.BlockSpec((8, 8), lambda i: (0, 0)),
        ],
        out_specs=pl.BlockSpec((8, 128), lambda i: (0, 0)),
        out_shape=jax.ShapeDtypeStruct((8, 128), jnp.float32),
    )(xgp, jnp.pad(p, ((0, 0), (0, 5))), pa)


def _p1_body(xg_ref, p_ref, x_ref, dist_ref, wqt_ref, wkt_ref, wvt_ref,
             bq_ref, bk_ref, bv_ref, pa_ref, wp2t_ref, bp2_ref,
             w_ref, veff_ref, acc_ref):
    i = pl.program_id(0)
    xg = xg_ref[...]
    xx = x_ref[...]
    e2 = (xg[:, :, :C] - xx[:, None, :]).reshape(ER, C)
    kk = jnp.dot(e2, wkt_ref[...],
                 preferred_element_type=jnp.float32) + bk_ref[...]
    vv = jnp.dot(e2, wvt_ref[...],
                 preferred_element_type=jnp.float32) + bv_ref[...]
    q = jnp.dot(xx, wqt_ref[...],
                preferred_element_type=jnp.float32) + bq_ref[...]

    pa = pa_ref[...]
    pr3 = xg[:, :, C:C + 3] - p_ref[...][:, None, :3]
    t = pa[3:4, :3].reshape(1, 1, 3)
    for j in range(3):
        t = t + pr3[:, :, j:j + 1] * pa[j:j + 1, :3].reshape(1, 1, 3)
    t = jnp.maximum(t, 0.0)
    pr = bp2_ref[...][None]
    for j in range(3):
        pr = pr + t[:, :, j:j + 1] * wp2t_ref[j:j + 1, :][None]
    w3 = q[:, None, :] - kk.reshape(NB, NSAMP, C) + pr
    dw = jnp.exp(-dist_ref[...])[:, :, None]
    veff = vv.reshape(NB, NSAMP, C) * dw + pr
    w_ref[...] = w3
    veff_ref[...] = veff
    sw = jnp.sum(w3, axis=(0, 1)).reshape(1, C)
    swsq = jnp.sum(w3 * w3, axis=(0, 1)).reshape(1, C)
    upd = jnp.concatenate([sw, swsq, jnp.zeros((6, C), jnp.float32)], axis=0)

    @pl.when(i == 0)
    def _():
        acc_ref[...] = upd

    @pl.when(i > 0)
    def _():
        acc_ref[...] += upd


def _p1(xg3, p, x, dist, params, pa):
    full = lambda r, c: pl.BlockSpec((r, c), lambda i: (0, 0))
    return pl.pallas_call(
        _p1_body,
        grid=(NBLK,),
        in_specs=[
            pl.BlockSpec((NB, NSAMP, TW), lambda i: (i, 0, 0)),
            pl.BlockSpec((NB, 8), lambda i: (i, 0)),
            pl.BlockSpec((NB, C), lambda i: (i, 0)),
            pl.BlockSpec((NB, NSAMP), lambda i: (i, 0)),
            full(C, C), full(C, C), full(C, C),
            full(1, C), full(1, C), full(1, C),
            full(8, 8), full(8, C), full(1, C),
        ],
        out_specs=[
            pl.BlockSpec((NB, NSAMP, C), lambda i: (i, 0, 0)),
            pl.BlockSpec((NB, NSAMP, C), lambda i: (i, 0, 0)),
            pl.BlockSpec((8, C), lambda i: (0, 0)),
        ],
        out_shape=[
            jax.ShapeDtypeStruct((N, NSAMP, C), jnp.float32),
            jax.ShapeDtypeStruct((N, NSAMP, C), jnp.float32),
            jax.ShapeDtypeStruct((8, C), jnp.float32),
        ],
    )(xg3, jnp.pad(p, ((0, 0), (0, 5))), x, dist,
      params['Wq'].T, params['Wk'].T, params['Wv'].T,
      params['bq'][None], params['bk'][None], params['bv'][None],
      pa, jnp.pad(params['Wp2'].T, ((0, 5), (0, 0))), params['bp2'][None])


def _p2_body(w_ref, aff_ref, wl1t_ref, bl1_ref, u_ref, acc_ref):
    i = pl.program_id(0)
    a = aff_ref[...]
    w = w_ref[...].reshape(ER, C)
    wb = jnp.maximum(w * a[0:1, :] + a[1:2, :], 0.0)
    u = jnp.dot(wb, wl1t_ref[...],
                preferred_element_type=jnp.float32) + bl1_ref[...]
    u_ref[...] = u.reshape(NB, NSAMP, CG)
    su = jnp.sum(u, axis=0).reshape(1, CG)
    susq = jnp.sum(u * u, axis=0).reshape(1, CG)
    upd = jnp.concatenate([su, susq, jnp.zeros((6, CG), jnp.float32)], axis=0)

    @pl.when(i == 0)
    def _():
        acc_ref[...] = upd

    @pl.when(i > 0)
    def _():
        acc_ref[...] += upd


def _p2(w, aff1, params):
    full = lambda r, c: pl.BlockSpec((r, c), lambda i: (0, 0))
    return pl.pallas_call(
        _p2_body,
        grid=(NBLK,),
        in_specs=[
            pl.BlockSpec((NB, NSAMP, C), lambda i: (i, 0, 0)),
            full(8, C), full(C, CG), full(1, CG),
        ],
        out_specs=[
            pl.BlockSpec((NB, NSAMP, CG), lambda i: (i, 0, 0)),
            pl.BlockSpec((8, CG), lambda i: (0, 0)),
        ],
        out_shape=[
            jax.ShapeDtypeStruct((N, NSAMP, CG), jnp.float32),
            jax.ShapeDtypeStruct((8, CG), jnp.float32),
        ],
    )(w, aff1, params['Wl1'].T, params['bl1'][None])


def _p3_body(u_ref, veff_ref, aff_ref, wl2t_ref, bl2_ref, out_ref):
    a = aff_ref[...]
    u = u_ref[...].reshape(ER, CG)
    ub = jnp.maximum(u * a[0:1, :] + a[1:2, :], 0.0)
    s = jnp.dot(ub, wl2t_ref[...],
                preferred_element_type=jnp.float32) + bl2_ref[...]
    s3 = s.reshape(NB, NSAMP, CG)
    mx = jnp.max(s3, axis=1, keepdims=True)
    es = jnp.exp(s3 - mx)
    sm = es / jnp.sum(es, axis=1, keepdims=True)
    v3 = veff_ref[...]
    outs = []
    for g in range(SHARE):
        outs.append(jnp.sum(v3[:, :, g * CG:(g + 1) * CG] * sm, axis=1))
    out_ref[...] = jnp.concatenate(outs, axis=1)


def _p3(u, veff, aff2, params):
    full = lambda r, c: pl.BlockSpec((r, c), lambda i: (0, 0))
    return pl.pallas_call(
        _p3_body,
        grid=(NBLK,),
        in_specs=[
            pl.BlockSpec((NB, NSAMP, CG), lambda i: (i, 0, 0)),
            pl.BlockSpec((NB, NSAMP, C), lambda i: (i, 0, 0)),
            full(8, CG), full(CG, CG), full(1, CG),
        ],
        out_specs=pl.BlockSpec((NB, C), lambda i: (i, 0)),
        out_shape=jax.ShapeDtypeStruct((N, C), jnp.float32),
    )(u, veff, aff2, params['Wl2'].T, params['bl2'][None])


def _bn_affine(ssum, ssq, cnt, gamma, beta):
    mean = ssum / cnt
    var = ssq / cnt - mean * mean
    scale = gamma / jnp.sqrt(var + EPS)
    return scale, beta - mean * scale


def kernel(p, n, x, o, params):
    idx, dist = _knn(p)
    table = jnp.concatenate([x, jnp.pad(p, ((0, 0), (0, 13)))], axis=1)
    xg3 = _gather(table, idx.reshape(-1)).reshape(N, NSAMP, TW)

    cnt = jnp.float32(N * NSAMP)
    pa0 = jnp.zeros((8, 8), jnp.float32)
    pa0 = pa0.at[:3, :3].set(params['Wp1'].T)
    pa0 = pa0.at[3, :3].set(params['bp1'])
    acc0 = _p0(xg3[:, :, C:], p, pa0)
    sc_p, sh_p = _bn_affine(acc0[0, :3], acc0[1, :3], cnt,
                            params['gp'], params['bp'])
    pa = jnp.zeros((8, 8), jnp.float32)
    pa = pa.at[:3, :3].set(params['Wp1'].T * sc_p[None, :])
    pa = pa.at[3, :3].set(params['bp1'] * sc_p + sh_p)

    w, veff, acc1 = _p1(xg3, p, x, dist, params, pa)
    sc1, sh1 = _bn_affine(acc1[0], acc1[1], cnt, params['g1'], params['b1'])
    aff1 = jnp.concatenate([sc1[None], sh1[None],
                            jnp.zeros((6, C), jnp.float32)], axis=0)

    u, acc2 = _p2(w, aff1, params)
    sc2, sh2 = _bn_affine(acc2[0], acc2[1], cnt, params['g2'], params['b2'])
    aff2 = jnp.concatenate([sc2[None], sh2[None],
                            jnp.zeros((6, CG), jnp.float32)], axis=0)

    return _p3(u, veff, aff2, params)


# speedup vs baseline: 3.6014x; 1.0539x over previous
import functools

import jax
import jax.numpy as jnp
from jax import lax
from jax.experimental import pallas as pl
from jax.experimental.pallas import tpu as pltpu
from jax.experimental.pallas import tpu_sc as plsc

N = 10000
NSAMP = 16
C = 128
SHARE = 8
CG = C // SHARE
TW = C + 16
EPS = 1e-5

KPAD = 10240
QB = 512
NQB = KPAD // QB

NB = 200
NBLK = N // NB
ER = NB * NSAMP

BIGF = 3.0e38
BIGI = 2 ** 30


def _knn_body(q_ref, kt_ref, qn_ref, kn_ref, idx_ref, dist_ref):
    cross = jnp.dot(q_ref[...], kt_ref[...],
                    preferred_element_type=jnp.float32)
    d2 = (qn_ref[...][:, 0:1] + kn_ref[...][0:1, :]) - 2.0 * cross
    idxs, dists = [], []
    for _ in range(NSAMP):
        m = jnp.min(d2, axis=1, keepdims=True)
        sel = jnp.argmin(d2, axis=1).astype(jnp.int32)[:, None]
        idxs.append(sel)
        dists.append(jnp.sqrt(jnp.maximum(m, 0.0)))
        iot = lax.broadcasted_iota(jnp.int32, (QB, KPAD), 1)
        d2 = jnp.where(iot == sel, BIGF, d2)
    idx_ref[...] = jnp.concatenate(idxs, axis=1)
    dist_ref[...] = jnp.concatenate(dists, axis=1)


def _knn(p):
    qpad = jnp.zeros((KPAD, 8), dtype=jnp.float32)
    qpad = qpad.at[:N, :3].set(p)
    qpad = qpad.at[N:, :3].set(1e15)
    kt = qpad.T[:8, :]
    nrm = jnp.sum(qpad[:, :3] * qpad[:, :3], axis=1)
    qn = jnp.broadcast_to(nrm[:, None], (KPAD, 8))
    kn = jnp.broadcast_to(nrm[None, :], (8, KPAD))
    idx, dist = pl.pallas_call(
        _knn_body,
        grid=(NQB,),
        in_specs=[
            pl.BlockSpec((QB, 8), lambda i: (i, 0)),
            pl.BlockSpec((8, KPAD), lambda i: (0, 0)),
            pl.BlockSpec((QB, 8), lambda i: (i, 0)),
            pl.BlockSpec((8, KPAD), lambda i: (0, 0)),
        ],
        out_specs=[
            pl.BlockSpec((QB, NSAMP), lambda i: (i, 0)),
            pl.BlockSpec((QB, NSAMP), lambda i: (i, 0)),
        ],
        out_shape=[
            jax.ShapeDtypeStruct((KPAD, NSAMP), jnp.int32),
            jax.ShapeDtypeStruct((KPAD, NSAMP), jnp.float32),
        ],
    )(qpad, kt, qn, kn)
    return idx[:N], dist[:N]


NW = 32
GCH = 128
GIT = 40
BPW = GCH * GIT
BPAD = NW * BPW


NRING = 4


def _gather(table, idx_flat):
    idx_flat = jnp.pad(idx_flat, (0, BPAD - N * NSAMP))
    mesh = plsc.VectorSubcoreMesh(core_axis_name="c", subcore_axis_name="s")

    @functools.partial(
        pl.kernel,
        mesh=mesh,
        compiler_params=pltpu.CompilerParams(use_tc_tiling_on_sc=False),
        out_type=jax.ShapeDtypeStruct((BPAD, TW), jnp.float32),
        scratch_types=[
            pltpu.VMEM((BPW,), jnp.int32),
        ] + [pltpu.VMEM((GCH, TW), jnp.float32) for _ in range(NRING)]
          + [pltpu.SemaphoreType.DMA for _ in range(2 * NRING)],
    )
    def gather_k(table_hbm, idx_hbm, out_hbm, idx_all, *bufs_sems):
        rows = bufs_sems[:NRING]
        gsem = bufs_sems[NRING:2 * NRING]
        wsem = bufs_sems[2 * NRING:]
        wid = lax.axis_index("s") * 2 + lax.axis_index("c")
        base = wid * BPW
        pltpu.sync_copy(idx_hbm.at[pl.ds(base, BPW)], idx_all)

        def body(t, carry):
            @pl.when(t > 0)
            def _():
                for b in range(NRING):
                    pltpu.make_async_copy(
                        rows[b], out_hbm.at[pl.ds(base, GCH)], wsem[b]).wait()

            for b in range(NRING):
                i = t * NRING + b
                pltpu.async_copy(
                    table_hbm.at[idx_all.at[pl.ds(i * GCH, GCH)]],
                    rows[b], gsem[b])
            for b in range(NRING):
                i = t * NRING + b
                pltpu.make_async_copy(
                    table_hbm.at[idx_all.at[pl.ds(i * GCH, GCH)]],
                    rows[b], gsem[b]).wait()
                pltpu.async_copy(
                    rows[b], out_hbm.at[pl.ds(base + i * GCH, GCH)], wsem[b])
            return carry

        lax.fori_loop(0, GIT // NRING, body, 0)
        for b in range(NRING):
            pltpu.make_async_copy(
                rows[b], out_hbm.at[pl.ds(base, GCH)], wsem[b]).wait()

    return gather_k(table, idx_flat)[:N * NSAMP]


def _p0_body(xgp_ref, p_ref, pa_ref, acc_ref):
    i = pl.program_id(0)
    pr3 = xgp_ref[...][:, :, :3] - p_ref[...][:, None, :3]
    pa = pa_ref[...]
    t = pa[3:4, :3].reshape(1, 1, 3)
    for j in range(3):
        t = t + pr3[:, :, j:j + 1] * pa[j:j + 1, :3].reshape(1, 1, 3)
    s1 = jnp.sum(t, axis=(0, 1)).reshape(1, 3)
    s2 = jnp.sum(t * t, axis=(0, 1)).reshape(1, 3)
    upd = jnp.concatenate(
        [jnp.pad(s1, ((0, 0), (0, 125))),
         jnp.pad(s2, ((0, 0), (0, 125))),
         jnp.zeros((6, 128), jnp.float32)], axis=0)

    @pl.when(i == 0)
    def _():
        acc_ref[...] = upd

    @pl.when(i > 0)
    def _():
        acc_ref[...] += upd


def _p0(xgp, p, pa):
    return pl.pallas_call(
        _p0_body,
        grid=(NBLK,),
        in_specs=[
            pl.BlockSpec((NB, NSAMP, 16), lambda i: (i, 0, 0)),
            pl.BlockSpec((NB, 8), lambda i: (i, 0)),
            pl.BlockSpec((8, 8), lambda i: (0, 0)),
        ],
        out_specs=pl.BlockSpec((8, 128), lambda i: (0, 0)),
        out_shape=jax.ShapeDtypeStruct((8, 128), jnp.float32),
    )(xgp, jnp.pad(p, ((0, 0), (0, 5))), pa)


def _p1_body(xg_ref, p_ref, x_ref, dist_ref, wqt_ref, wkt_ref, wvt_ref,
             bq_ref, bk_ref, bv_ref, pa_ref, wp2t_ref, bp2_ref,
             w_ref, veff_ref, acc_ref):
    i = pl.program_id(0)
    xg = xg_ref[...]
    xx = x_ref[...]
    e2 = (xg[:, :, :C] - xx[:, None, :]).reshape(ER, C)
    kk = jnp.dot(e2, wkt_ref[...],
                 preferred_element_type=jnp.float32) + bk_ref[...]
    vv = jnp.dot(e2, wvt_ref[...],
                 preferred_element_type=jnp.float32) + bv_ref[...]
    q = jnp.dot(xx, wqt_ref[...],
                preferred_element_type=jnp.float32) + bq_ref[...]

    pa = pa_ref[...]
    pr3 = xg[:, :, C:C + 3] - p_ref[...][:, None, :3]
    t = pa[3:4, :3].reshape(1, 1, 3)
    for j in range(3):
        t = t + pr3[:, :, j:j + 1] * pa[j:j + 1, :3].reshape(1, 1, 3)
    t = jnp.maximum(t, 0.0)
    pr = bp2_ref[...][None]
    for j in range(3):
        pr = pr + t[:, :, j:j + 1] * wp2t_ref[j:j + 1, :][None]
    w3 = q[:, None, :] - kk.reshape(NB, NSAMP, C) + pr
    dw = jnp.exp(-dist_ref[...])[:, :, None]
    veff = vv.reshape(NB, NSAMP, C) * dw + pr
    w_ref[...] = w3
    veff_ref[...] = veff
    sw = jnp.sum(w3, axis=(0, 1)).reshape(1, C)
    swsq = jnp.sum(w3 * w3, axis=(0, 1)).reshape(1, C)
    upd = jnp.concatenate([sw, swsq, jnp.zeros((6, C), jnp.float32)], axis=0)

    @pl.when(i == 0)
    def _():
        acc_ref[...] = upd

    @pl.when(i > 0)
    def _():
        acc_ref[...] += upd


def _p1(xg3, p, x, dist, params, pa):
    full = lambda r, c: pl.BlockSpec((r, c), lambda i: (0, 0))
    return pl.pallas_call(
        _p1_body,
        grid=(NBLK,),
        in_specs=[
            pl.BlockSpec((NB, NSAMP, TW), lambda i: (i, 0, 0)),
            pl.BlockSpec((NB, 8), lambda i: (i, 0)),
            pl.BlockSpec((NB, C), lambda i: (i, 0)),
            pl.BlockSpec((NB, NSAMP), lambda i: (i, 0)),
            full(C, C), full(C, C), full(C, C),
            full(1, C), full(1, C), full(1, C),
            full(8, 8), full(8, C), full(1, C),
        ],
        out_specs=[
            pl.BlockSpec((NB, NSAMP, C), lambda i: (i, 0, 0)),
            pl.BlockSpec((NB, NSAMP, C), lambda i: (i, 0, 0)),
            pl.BlockSpec((8, C), lambda i: (0, 0)),
        ],
        out_shape=[
            jax.ShapeDtypeStruct((N, NSAMP, C), jnp.float32),
            jax.ShapeDtypeStruct((N, NSAMP, C), jnp.float32),
            jax.ShapeDtypeStruct((8, C), jnp.float32),
        ],
    )(xg3, jnp.pad(p, ((0, 0), (0, 5))), x, dist,
      params['Wq'].T, params['Wk'].T, params['Wv'].T,
      params['bq'][None], params['bk'][None], params['bv'][None],
      pa, jnp.pad(params['Wp2'].T, ((0, 5), (0, 0))), params['bp2'][None])


def _p2_body(w_ref, aff_ref, wl1t_ref, bl1_ref, u_ref, acc_ref):
    i = pl.program_id(0)
    a = aff_ref[...]
    w = w_ref[...].reshape(ER, C)
    wb = jnp.maximum(w * a[0:1, :] + a[1:2, :], 0.0)
    u = jnp.dot(wb, wl1t_ref[...],
                preferred_element_type=jnp.float32) + bl1_ref[...]
    u_ref[...] = u.reshape(NB, NSAMP, CG)
    su = jnp.sum(u, axis=0).reshape(1, CG)
    susq = jnp.sum(u * u, axis=0).reshape(1, CG)
    upd = jnp.concatenate([su, susq, jnp.zeros((6, CG), jnp.float32)], axis=0)

    @pl.when(i == 0)
    def _():
        acc_ref[...] = upd

    @pl.when(i > 0)
    def _():
        acc_ref[...] += upd


def _p2(w, aff1, params):
    full = lambda r, c: pl.BlockSpec((r, c), lambda i: (0, 0))
    return pl.pallas_call(
        _p2_body,
        grid=(NBLK,),
        in_specs=[
            pl.BlockSpec((NB, NSAMP, C), lambda i: (i, 0, 0)),
            full(8, C), full(C, CG), full(1, CG),
        ],
        out_specs=[
            pl.BlockSpec((NB, NSAMP, CG), lambda i: (i, 0, 0)),
            pl.BlockSpec((8, CG), lambda i: (0, 0)),
        ],
        out_shape=[
            jax.ShapeDtypeStruct((N, NSAMP, CG), jnp.float32),
            jax.ShapeDtypeStruct((8, CG), jnp.float32),
        ],
    )(w, aff1, params['Wl1'].T, params['bl1'][None])


def _p3_body(u_ref, veff_ref, aff_ref, wl2t_ref, bl2_ref, out_ref):
    a = aff_ref[...]
    u = u_ref[...].reshape(ER, CG)
    ub = jnp.maximum(u * a[0:1, :] + a[1:2, :], 0.0)
    s = jnp.dot(ub, wl2t_ref[...],
                preferred_element_type=jnp.float32) + bl2_ref[...]
    s3 = s.reshape(NB, NSAMP, CG)
    mx = jnp.max(s3, axis=1, keepdims=True)
    es = jnp.exp(s3 - mx)
    sm = es / jnp.sum(es, axis=1, keepdims=True)
    v3 = veff_ref[...]
    outs = []
    for g in range(SHARE):
        outs.append(jnp.sum(v3[:, :, g * CG:(g + 1) * CG] * sm, axis=1))
    out_ref[...] = jnp.concatenate(outs, axis=1)


def _p3(u, veff, aff2, params):
    full = lambda r, c: pl.BlockSpec((r, c), lambda i: (0, 0))
    return pl.pallas_call(
        _p3_body,
        grid=(NBLK,),
        in_specs=[
            pl.BlockSpec((NB, NSAMP, CG), lambda i: (i, 0, 0)),
            pl.BlockSpec((NB, NSAMP, C), lambda i: (i, 0, 0)),
            full(8, CG), full(CG, CG), full(1, CG),
        ],
        out_specs=pl.BlockSpec((NB, C), lambda i: (i, 0)),
        out_shape=jax.ShapeDtypeStruct((N, C), jnp.float32),
    )(u, veff, aff2, params['Wl2'].T, params['bl2'][None])


def _bn_affine(ssum, ssq, cnt, gamma, beta):
    mean = ssum / cnt
    var = ssq / cnt - mean * mean
    scale = gamma / jnp.sqrt(var + EPS)
    return scale, beta - mean * scale


def kernel(p, n, x, o, params):
    idx, dist = _knn(p)
    table = jnp.concatenate([x, jnp.pad(p, ((0, 0), (0, 13)))], axis=1)
    xg3 = _gather(table, idx.reshape(-1)).reshape(N, NSAMP, TW)

    cnt = jnp.float32(N * NSAMP)
    pa0 = jnp.zeros((8, 8), jnp.float32)
    pa0 = pa0.at[:3, :3].set(params['Wp1'].T)
    pa0 = pa0.at[3, :3].set(params['bp1'])
    acc0 = _p0(xg3[:, :, C:], p, pa0)
    sc_p, sh_p = _bn_affine(acc0[0, :3], acc0[1, :3], cnt,
                            params['gp'], params['bp'])
    pa = jnp.zeros((8, 8), jnp.float32)
    pa = pa.at[:3, :3].set(params['Wp1'].T * sc_p[None, :])
    pa = pa.at[3, :3].set(params['bp1'] * sc_p + sh_p)

    w, veff, acc1 = _p1(xg3, p, x, dist, params, pa)
    sc1, sh1 = _bn_affine(acc1[0], acc1[1], cnt, params['g1'], params['b1'])
    aff1 = jnp.concatenate([sc1[None], sh1[None],
                            jnp.zeros((6, C), jnp.float32)], axis=0)

    u, acc2 = _p2(w, aff1, params)
    sc2, sh2 = _bn_affine(acc2[0], acc2[1], cnt, params['g2'], params['b2'])
    aff2 = jnp.concatenate([sc2[None], sh2[None],
                            jnp.zeros((6, CG), jnp.float32)], axis=0)

    return _p3(u, veff, aff2, params)


# speedup vs baseline: 3.7032x; 1.0283x over previous
import functools

import jax
import jax.numpy as jnp
from jax import lax
from jax.experimental import pallas as pl
from jax.experimental.pallas import tpu as pltpu
from jax.experimental.pallas import tpu_sc as plsc

N = 10000
NSAMP = 16
C = 128
SHARE = 8
CG = C // SHARE
TW = C + 16
EPS = 1e-5

KPAD = 10240
QB = 512
NQB = KPAD // QB

NB = 200
NBLK = N // NB
ER = NB * NSAMP

BIGF = 3.0e38
BIGI = 2 ** 30


def _knn_body(q_ref, kt_ref, qn_ref, kn_ref, idx_ref, dist_ref):
    cross = jnp.dot(q_ref[...], kt_ref[...],
                    preferred_element_type=jnp.float32)
    d2 = (qn_ref[...][:, 0:1] + kn_ref[...][0:1, :]) - 2.0 * cross
    idxs, dists = [], []
    for _ in range(NSAMP):
        m = jnp.min(d2, axis=1, keepdims=True)
        sel = jnp.argmin(d2, axis=1).astype(jnp.int32)[:, None]
        idxs.append(sel)
        dists.append(jnp.sqrt(jnp.maximum(m, 0.0)))
        iot = lax.broadcasted_iota(jnp.int32, (QB, KPAD), 1)
        d2 = jnp.where(iot == sel, BIGF, d2)
    idx_ref[...] = jnp.concatenate(idxs, axis=1)
    dist_ref[...] = jnp.concatenate(dists, axis=1)


def _knn(p):
    qpad = jnp.zeros((KPAD, 8), dtype=jnp.float32)
    qpad = qpad.at[:N, :3].set(p)
    qpad = qpad.at[N:, :3].set(1e15)
    kt = qpad.T[:8, :]
    nrm = jnp.sum(qpad[:, :3] * qpad[:, :3], axis=1)
    qn = jnp.broadcast_to(nrm[:, None], (KPAD, 8))
    kn = jnp.broadcast_to(nrm[None, :], (8, KPAD))
    idx, dist = pl.pallas_call(
        _knn_body,
        grid=(NQB,),
        in_specs=[
            pl.BlockSpec((QB, 8), lambda i: (i, 0)),
            pl.BlockSpec((8, KPAD), lambda i: (0, 0)),
            pl.BlockSpec((QB, 8), lambda i: (i, 0)),
            pl.BlockSpec((8, KPAD), lambda i: (0, 0)),
        ],
        out_specs=[
            pl.BlockSpec((QB, NSAMP), lambda i: (i, 0)),
            pl.BlockSpec((QB, NSAMP), lambda i: (i, 0)),
        ],
        out_shape=[
            jax.ShapeDtypeStruct((KPAD, NSAMP), jnp.int32),
            jax.ShapeDtypeStruct((KPAD, NSAMP), jnp.float32),
        ],
    )(qpad, kt, qn, kn)
    return idx[:N], dist[:N]


NW = 32
GCH = 128
GIT = 40
BPW = GCH * GIT
BPAD = NW * BPW


NRING = 4


def _gather(table, idx_flat):
    idx_flat = jnp.pad(idx_flat, (0, BPAD - N * NSAMP))
    mesh = plsc.VectorSubcoreMesh(core_axis_name="c", subcore_axis_name="s")

    @functools.partial(
        pl.kernel,
        mesh=mesh,
        compiler_params=pltpu.CompilerParams(use_tc_tiling_on_sc=False),
        out_type=jax.ShapeDtypeStruct((BPAD, TW), jnp.float32),
        scratch_types=[
            pltpu.VMEM((BPW,), jnp.int32),
        ] + [pltpu.VMEM((GCH, TW), jnp.float32) for _ in range(NRING)]
          + [pltpu.SemaphoreType.DMA for _ in range(2 * NRING)],
    )
    def gather_k(table_hbm, idx_hbm, out_hbm, idx_all, *bufs_sems):
        rows = bufs_sems[:NRING]
        gsem = bufs_sems[NRING:2 * NRING]
        wsem = bufs_sems[2 * NRING:]
        wid = lax.axis_index("s") * 2 + lax.axis_index("c")
        base = wid * BPW
        pltpu.sync_copy(idx_hbm.at[pl.ds(base, BPW)], idx_all)

        def body(t, carry):
            @pl.when(t > 0)
            def _():
                for b in range(NRING):
                    pltpu.make_async_copy(
                        rows[b], out_hbm.at[pl.ds(base, GCH)], wsem[b]).wait()

            for b in range(NRING):
                i = t * NRING + b
                pltpu.async_copy(
                    table_hbm.at[idx_all.at[pl.ds(i * GCH, GCH)]],
                    rows[b], gsem[b])
            for b in range(NRING):
                i = t * NRING + b
                pltpu.make_async_copy(
                    table_hbm.at[idx_all.at[pl.ds(i * GCH, GCH)]],
                    rows[b], gsem[b]).wait()
                pltpu.async_copy(
                    rows[b], out_hbm.at[pl.ds(base + i * GCH, GCH)], wsem[b])
            return carry

        lax.fori_loop(0, GIT // NRING, body, 0)
        for b in range(NRING):
            pltpu.make_async_copy(
                rows[b], out_hbm.at[pl.ds(base, GCH)], wsem[b]).wait()

    return gather_k(table, idx_flat)


def _p0_body(xgp_ref, p_ref, pa_ref, acc_ref):
    i = pl.program_id(0)
    pr3 = xgp_ref[...][:, :, :3] - p_ref[...][:, None, :3]
    pa = pa_ref[...]
    t = pa[3:4, :3].reshape(1, 1, 3)
    for j in range(3):
        t = t + pr3[:, :, j:j + 1] * pa[j:j + 1, :3].reshape(1, 1, 3)
    s1 = jnp.sum(t, axis=(0, 1)).reshape(1, 3)
    s2 = jnp.sum(t * t, axis=(0, 1)).reshape(1, 3)
    upd = jnp.concatenate(
        [jnp.pad(s1, ((0, 0), (0, 125))),
         jnp.pad(s2, ((0, 0), (0, 125))),
         jnp.zeros((6, 128), jnp.float32)], axis=0)

    @pl.when(i == 0)
    def _():
        acc_ref[...] = upd

    @pl.when(i > 0)
    def _():
        acc_ref[...] += upd


def _p0(xgp, p, pa):
    return pl.pallas_call(
        _p0_body,
        grid=(NBLK,),
        in_specs=[
            pl.BlockSpec((NB, NSAMP, 16), lambda i: (i, 0, 0)),
            pl.BlockSpec((NB, 8), lambda i: (i, 0)),
            pl.BlockSpec((8, 8), lambda i: (0, 0)),
        ],
        out_specs=pl.BlockSpec((8, 128), lambda i: (0, 0)),
        out_shape=jax.ShapeDtypeStruct((8, 128), jnp.float32),
    )(xgp, jnp.pad(p, ((0, 0), (0, 5))), pa)


def _p1_body(xg_ref, p_ref, x_ref, dist_ref, wqt_ref, wkt_ref, wvt_ref,
             bq_ref, bk_ref, bv_ref, pa_ref, wp2t_ref, bp2_ref,
             w_ref, veff_ref, acc_ref):
    i = pl.program_id(0)
    xg = xg_ref[...]
    xx = x_ref[...]
    e2 = (xg[:, :, :C] - xx[:, None, :]).reshape(ER, C)
    kk = jnp.dot(e2, wkt_ref[...],
                 preferred_element_type=jnp.float32) + bk_ref[...]
    vv = jnp.dot(e2, wvt_ref[...],
                 preferred_element_type=jnp.float32) + bv_ref[...]
    q = jnp.dot(xx, wqt_ref[...],
                preferred_element_type=jnp.float32) + bq_ref[...]

    pa = pa_ref[...]
    pr3 = xg[:, :, C:C + 3] - p_ref[...][:, None, :3]
    t = pa[3:4, :3].reshape(1, 1, 3)
    for j in range(3):
        t = t + pr3[:, :, j:j + 1] * pa[j:j + 1, :3].reshape(1, 1, 3)
    t = jnp.maximum(t, 0.0)
    pr = bp2_ref[...][None]
    for j in range(3):
        pr = pr + t[:, :, j:j + 1] * wp2t_ref[j:j + 1, :][None]
    w3 = q[:, None, :] - kk.reshape(NB, NSAMP, C) + pr
    dw = jnp.exp(-dist_ref[...])[:, :, None]
    veff = vv.reshape(NB, NSAMP, C) * dw + pr
    w_ref[...] = w3
    veff_ref[...] = veff
    sw = jnp.sum(w3, axis=(0, 1)).reshape(1, C)
    swsq = jnp.sum(w3 * w3, axis=(0, 1)).reshape(1, C)
    upd = jnp.concatenate([sw, swsq, jnp.zeros((6, C), jnp.float32)], axis=0)

    @pl.when(i == 0)
    def _():
        acc_ref[...] = upd

    @pl.when(i > 0)
    def _():
        acc_ref[...] += upd


def _p1(xg3, p, x, dist, params, pa):
    full = lambda r, c: pl.BlockSpec((r, c), lambda i: (0, 0))
    return pl.pallas_call(
        _p1_body,
        grid=(NBLK,),
        in_specs=[
            pl.BlockSpec((NB, NSAMP, TW), lambda i: (i, 0, 0)),
            pl.BlockSpec((NB, 8), lambda i: (i, 0)),
            pl.BlockSpec((NB, C), lambda i: (i, 0)),
            pl.BlockSpec((NB, NSAMP), lambda i: (i, 0)),
            full(C, C), full(C, C), full(C, C),
            full(1, C), full(1, C), full(1, C),
            full(8, 8), full(8, C), full(1, C),
        ],
        out_specs=[
            pl.BlockSpec((NB, NSAMP, C), lambda i: (i, 0, 0)),
            pl.BlockSpec((NB, NSAMP, C), lambda i: (i, 0, 0)),
            pl.BlockSpec((8, C), lambda i: (0, 0)),
        ],
        out_shape=[
            jax.ShapeDtypeStruct((N, NSAMP, C), jnp.float32),
            jax.ShapeDtypeStruct((N, NSAMP, C), jnp.float32),
            jax.ShapeDtypeStruct((8, C), jnp.float32),
        ],
    )(xg3, jnp.pad(p, ((0, 0), (0, 5))), x, dist,
      params['Wq'].T, params['Wk'].T, params['Wv'].T,
      params['bq'][None], params['bk'][None], params['bv'][None],
      pa, jnp.pad(params['Wp2'].T, ((0, 5), (0, 0))), params['bp2'][None])


def _p2_body(w_ref, aff_ref, wl1t_ref, bl1_ref, u_ref, acc_ref):
    i = pl.program_id(0)
    a = aff_ref[...]
    w = w_ref[...].reshape(ER, C)
    wb = jnp.maximum(w * a[0:1, :] + a[1:2, :], 0.0)
    u = jnp.dot(wb, wl1t_ref[...],
                preferred_element_type=jnp.float32) + bl1_ref[...]
    u_ref[...] = u.reshape(NB, NSAMP, CG)
    su = jnp.sum(u, axis=0).reshape(1, CG)
    susq = jnp.sum(u * u, axis=0).reshape(1, CG)
    upd = jnp.concatenate([su, susq, jnp.zeros((6, CG), jnp.float32)], axis=0)

    @pl.when(i == 0)
    def _():
        acc_ref[...] = upd

    @pl.when(i > 0)
    def _():
        acc_ref[...] += upd


def _p2(w, aff1, params):
    full = lambda r, c: pl.BlockSpec((r, c), lambda i: (0, 0))
    return pl.pallas_call(
        _p2_body,
        grid=(NBLK,),
        in_specs=[
            pl.BlockSpec((NB, NSAMP, C), lambda i: (i, 0, 0)),
            full(8, C), full(C, CG), full(1, CG),
        ],
        out_specs=[
            pl.BlockSpec((NB, NSAMP, CG), lambda i: (i, 0, 0)),
            pl.BlockSpec((8, CG), lambda i: (0, 0)),
        ],
        out_shape=[
            jax.ShapeDtypeStruct((N, NSAMP, CG), jnp.float32),
            jax.ShapeDtypeStruct((8, CG), jnp.float32),
        ],
    )(w, aff1, params['Wl1'].T, params['bl1'][None])


def _p3_body(u_ref, veff_ref, aff_ref, wl2t_ref, bl2_ref, out_ref):
    a = aff_ref[...]
    u = u_ref[...].reshape(ER, CG)
    ub = jnp.maximum(u * a[0:1, :] + a[1:2, :], 0.0)
    s = jnp.dot(ub, wl2t_ref[...],
                preferred_element_type=jnp.float32) + bl2_ref[...]
    s3 = s.reshape(NB, NSAMP, CG)
    mx = jnp.max(s3, axis=1, keepdims=True)
    es = jnp.exp(s3 - mx)
    sm = es / jnp.sum(es, axis=1, keepdims=True)
    v3 = veff_ref[...]
    outs = []
    for g in range(SHARE):
        outs.append(jnp.sum(v3[:, :, g * CG:(g + 1) * CG] * sm, axis=1))
    out_ref[...] = jnp.concatenate(outs, axis=1)


def _p3(u, veff, aff2, params):
    full = lambda r, c: pl.BlockSpec((r, c), lambda i: (0, 0))
    return pl.pallas_call(
        _p3_body,
        grid=(NBLK,),
        in_specs=[
            pl.BlockSpec((NB, NSAMP, CG), lambda i: (i, 0, 0)),
            pl.BlockSpec((NB, NSAMP, C), lambda i: (i, 0, 0)),
            full(8, CG), full(CG, CG), full(1, CG),
        ],
        out_specs=pl.BlockSpec((NB, C), lambda i: (i, 0)),
        out_shape=jax.ShapeDtypeStruct((N, C), jnp.float32),
    )(u, veff, aff2, params['Wl2'].T, params['bl2'][None])


def _bn_affine(ssum, ssq, cnt, gamma, beta):
    mean = ssum / cnt
    var = ssq / cnt - mean * mean
    scale = gamma / jnp.sqrt(var + EPS)
    return scale, beta - mean * scale


def kernel(p, n, x, o, params):
    idx, dist = _knn(p)
    table = jnp.concatenate([x, jnp.pad(p, ((0, 0), (0, 13)))], axis=1)
    xg3 = _gather(table, idx.reshape(-1)).reshape(BPAD // NSAMP, NSAMP, TW)

    cnt = jnp.float32(N * NSAMP)
    pa0 = jnp.zeros((8, 8), jnp.float32)
    pa0 = pa0.at[:3, :3].set(params['Wp1'].T)
    pa0 = pa0.at[3, :3].set(params['bp1'])
    acc0 = _p0(xg3[:N, :, C:], p, pa0)
    sc_p, sh_p = _bn_affine(acc0[0, :3], acc0[1, :3], cnt,
                            params['gp'], params['bp'])
    pa = jnp.zeros((8, 8), jnp.float32)
    pa = pa.at[:3, :3].set(params['Wp1'].T * sc_p[None, :])
    pa = pa.at[3, :3].set(params['bp1'] * sc_p + sh_p)

    w, veff, acc1 = _p1(xg3, p, x, dist, params, pa)
    sc1, sh1 = _bn_affine(acc1[0], acc1[1], cnt, params['g1'], params['b1'])
    aff1 = jnp.concatenate([sc1[None], sh1[None],
                            jnp.zeros((6, C), jnp.float32)], axis=0)

    u, acc2 = _p2(w, aff1, params)
    sc2, sh2 = _bn_affine(acc2[0], acc2[1], cnt, params['g2'], params['b2'])
    aff2 = jnp.concatenate([sc2[None], sh2[None],
                            jnp.zeros((6, CG), jnp.float32)], axis=0)

    return _p3(u, veff, aff2, params)
